# baseline (speedup 1.0000x reference)
import jax
import jax.numpy as jnp
from jax import lax
from jax.experimental import pallas as pl
from jax.experimental.pallas import tpu as pltpu
from jax.experimental.pallas import tpu_sc as plsc

_N = 10000
_M = 7000
_RB = 8
_NW = 32
_BLOCK = 1000


def _cls_feature(p):
    ids = jax.lax.broadcasted_iota(jnp.int32, p.shape, 1)
    maxv = jnp.max(p, axis=1, keepdims=True)
    return jnp.min(jnp.where(p == maxv, ids, p.shape[1]), axis=1).astype(jnp.float32)


def _linear_block(msg, h, e, p, wh_ref, wm_ref, we_ref, wc_ref, b_ref):
    cls = _cls_feature(p)
    acc = jnp.dot(h.astype(jnp.bfloat16), wh_ref[...],
                  preferred_element_type=jnp.float32)
    acc = acc + jnp.dot(msg.astype(jnp.bfloat16), wm_ref[...],
                        preferred_element_type=jnp.float32)
    acc = acc + jnp.dot(e.astype(jnp.bfloat16), we_ref[...],
                        preferred_element_type=jnp.float32)
    acc = acc + cls[:, None] * wc_ref[...] + b_ref[...]
    return jnp.maximum(acc, 0.0)


def _tc1_body(mb_ref, h_ref, e_ref, p_ref, wh_ref, wm_ref, we_ref, wc_ref,
              b_ref, out_ref):
    msg = jnp.sum(mb_ref[...], axis=1)
    out_ref[...] = _linear_block(msg, h_ref[...], e_ref[...], p_ref[...],
                                 wh_ref, wm_ref, we_ref, wc_ref, b_ref)


def _tc2_body(msg_ref, h_ref, e_ref, p_ref, wh_ref, wm_ref, we_ref, wc_ref,
              b_ref, prev_ref, out_ref):
    del prev_ref
    out_ref[...] = _linear_block(msg_ref[...], h_ref[...], e_ref[...],
                                 p_ref[...], wh_ref, wm_ref, we_ref, wc_ref,
                                 b_ref)


def _sc_sum_body(mb_hbm, msg_hbm, in0, in1, outbuf, sem0, sem1):
    K = mb_hbm.shape[1]
    n_chunk = mb_hbm.shape[2] // 16
    wid = lax.axis_index("s") * 2 + lax.axis_index("c")
    nblk = (_N - _M) // _RB
    nsteps = (nblk + _NW - 1) // _NW
    npairs = (nsteps + 1) // 2

    def blk_of(s):
        return s * _NW + wid

    def issue(s, buf, sem):
        @pl.when(blk_of(s) < nblk)
        def _():
            pltpu.async_copy(
                mb_hbm.at[pl.ds(_M + blk_of(s) * _RB, _RB)], buf, sem)

    def wait(s, buf, sem):
        @pl.when(blk_of(s) < nblk)
        def _():
            pltpu.make_async_copy(
                mb_hbm.at[pl.ds(_M, _RB)], buf, sem).wait()

    def compute(s, buf):
        blk = blk_of(s)

        @pl.when(blk < nblk)
        def _():
            def row(r, c2):
                for c in range(n_chunk):
                    acc = buf[r, 0, pl.ds(c * 16, 16)]
                    for k in range(1, K):
                        acc = acc + buf[r, k, pl.ds(c * 16, 16)]
                    outbuf[r, pl.ds(c * 16, 16)] = acc
                return c2

            lax.fori_loop(0, _RB, row, 0)
            pltpu.sync_copy(outbuf, msg_hbm.at[pl.ds(blk * _RB, _RB)])

    issue(0, in0, sem0)

    def pair(s2, carry):
        s_a = 2 * s2
        issue(s_a + 1, in1, sem1)
        wait(s_a, in0, sem0)
        compute(s_a, in0)
        issue(s_a + 2, in0, sem0)
        wait(s_a + 1, in1, sem1)
        compute(s_a + 1, in1)
        return carry

    lax.fori_loop(0, npairs, pair, 0)


def _sc_mailbox_sum(mailbox):
    K, d_msg = mailbox.shape[1], mailbox.shape[2]
    return pl.kernel(
        _sc_sum_body,
        out_type=jax.ShapeDtypeStruct((_N - _M, d_msg), jnp.float32),
        mesh=plsc.VectorSubcoreMesh(core_axis_name="c", subcore_axis_name="s"),
        scratch_types=[
            pltpu.VMEM((_RB, K, d_msg), jnp.float32),
            pltpu.VMEM((_RB, K, d_msg), jnp.float32),
            pltpu.VMEM((_RB, d_msg), jnp.float32),
            pltpu.SemaphoreType.DMA,
            pltpu.SemaphoreType.DMA,
        ],
    )(mailbox)


def kernel(mailbox_edge_message, node_hidden_rep, node_type_embedding, node_prediction, W, b):
    N, K, d_msg = mailbox_edge_message.shape
    d_h = node_hidden_rep.shape[1]
    d_e = node_type_embedding.shape[1]
    C = node_prediction.shape[1]
    d_out = W.shape[0]
    Wt = W.T
    wh = Wt[:d_h].astype(jnp.bfloat16)
    wm = Wt[d_h:d_h + d_msg].astype(jnp.bfloat16)
    we = Wt[d_h + d_msg:d_h + d_msg + d_e].astype(jnp.bfloat16)
    wc = Wt[d_h + d_msg + d_e:]
    b2 = b[None, :]

    w_specs = [
        pl.BlockSpec((d_h, d_out), lambda i: (0, 0)),
        pl.BlockSpec((d_msg, d_out), lambda i: (0, 0)),
        pl.BlockSpec((d_e, d_out), lambda i: (0, 0)),
        pl.BlockSpec((1, d_out), lambda i: (0, 0)),
        pl.BlockSpec((1, d_out), lambda i: (0, 0)),
    ]

    out1 = pl.pallas_call(
        _tc1_body,
        grid=(_M // _BLOCK,),
        in_specs=[
            pl.BlockSpec((_BLOCK, K, d_msg), lambda i: (i, 0, 0)),
            pl.BlockSpec((_BLOCK, d_h), lambda i: (i, 0)),
            pl.BlockSpec((_BLOCK, d_e), lambda i: (i, 0)),
            pl.BlockSpec((_BLOCK, C), lambda i: (i, 0)),
        ] + w_specs,
        out_specs=pl.BlockSpec((_BLOCK, d_out), lambda i: (i, 0)),
        out_shape=jax.ShapeDtypeStruct((N, d_out), jnp.float32),
        compiler_params=pltpu.CompilerParams(
            dimension_semantics=("arbitrary",),
        ),
    )(mailbox_edge_message, node_hidden_rep, node_type_embedding,
      node_prediction, wh, wm, we, wc, b2)

    msg_tail = _sc_mailbox_sum(mailbox_edge_message)

    m_blocks = _M // _BLOCK
    return pl.pallas_call(
        _tc2_body,
        grid=((N - _M) // _BLOCK,),
        in_specs=[
            pl.BlockSpec((_BLOCK, d_msg), lambda j: (j, 0)),
            pl.BlockSpec((_BLOCK, d_h), lambda j: (m_blocks + j, 0)),
            pl.BlockSpec((_BLOCK, d_e), lambda j: (m_blocks + j, 0)),
            pl.BlockSpec((_BLOCK, C), lambda j: (m_blocks + j, 0)),
        ] + w_specs + [
            pl.BlockSpec(memory_space=pltpu.MemorySpace.HBM),
        ],
        out_specs=pl.BlockSpec((_BLOCK, d_out), lambda j: (m_blocks + j, 0)),
        out_shape=jax.ShapeDtypeStruct((N, d_out), jnp.float32),
        input_output_aliases={9: 0},
        compiler_params=pltpu.CompilerParams(
            dimension_semantics=("arbitrary",),
        ),
    )(msg_tail, node_hidden_rep, node_type_embedding, node_prediction,
      wh, wm, we, wc, b2, out1)


# speedup vs baseline: 1.2695x; 1.2695x over previous
import jax
import jax.numpy as jnp
from jax.experimental import pallas as pl
from jax.experimental.pallas import tpu as pltpu

_BLOCK = 1000


def _cls_feature(p):
    ids = jax.lax.broadcasted_iota(jnp.int32, p.shape, 1)
    maxv = jnp.max(p, axis=1, keepdims=True)
    return jnp.min(jnp.where(p == maxv, ids, p.shape[1]), axis=1).astype(jnp.float32)


def _fused_body(mb_ref, h_ref, e_ref, p_ref, wh_ref, wm_ref, we_ref, wc_ref,
                b_ref, out_ref):
    msg = jnp.sum(mb_ref[...], axis=1)
    cls = _cls_feature(p_ref[...])
    acc = jnp.dot(h_ref[...].astype(jnp.bfloat16), wh_ref[...],
                  preferred_element_type=jnp.float32)
    acc = acc + jnp.dot(msg.astype(jnp.bfloat16), wm_ref[...],
                        preferred_element_type=jnp.float32)
    acc = acc + jnp.dot(e_ref[...].astype(jnp.bfloat16), we_ref[...],
                        preferred_element_type=jnp.float32)
    acc = acc + cls[:, None] * wc_ref[...] + b_ref[...]
    out_ref[...] = jnp.maximum(acc, 0.0)


def kernel(mailbox_edge_message, node_hidden_rep, node_type_embedding, node_prediction, W, b):
    N, K, d_msg = mailbox_edge_message.shape
    d_h = node_hidden_rep.shape[1]
    d_e = node_type_embedding.shape[1]
    d_out = W.shape[0]
    Wt = W.T
    wh = Wt[:d_h].astype(jnp.bfloat16)
    wm = Wt[d_h:d_h + d_msg].astype(jnp.bfloat16)
    we = Wt[d_h + d_msg:d_h + d_msg + d_e].astype(jnp.bfloat16)
    wc = Wt[d_h + d_msg + d_e:]
    b2 = b[None, :]

    return pl.pallas_call(
        _fused_body,
        grid=(N // _BLOCK,),
        in_specs=[
            pl.BlockSpec((_BLOCK, K, d_msg), lambda i: (i, 0, 0)),
            pl.BlockSpec((_BLOCK, d_h), lambda i: (i, 0)),
            pl.BlockSpec((_BLOCK, d_e), lambda i: (i, 0)),
            pl.BlockSpec((_BLOCK, node_prediction.shape[1]), lambda i: (i, 0)),
            pl.BlockSpec((d_h, d_out), lambda i: (0, 0)),
            pl.BlockSpec((d_msg, d_out), lambda i: (0, 0)),
            pl.BlockSpec((d_e, d_out), lambda i: (0, 0)),
            pl.BlockSpec((1, d_out), lambda i: (0, 0)),
            pl.BlockSpec((1, d_out), lambda i: (0, 0)),
        ],
        out_specs=pl.BlockSpec((_BLOCK, d_out), lambda i: (i, 0)),
        out_shape=jax.ShapeDtypeStruct((N, d_out), jnp.float32),
        compiler_params=pltpu.CompilerParams(
            dimension_semantics=("arbitrary",),
        ),
    )(mailbox_edge_message, node_hidden_rep, node_type_embedding,
      node_prediction, wh, wm, we, wc, b2)
